# baseline (device time: 22909 ns/iter reference)
import jax
import jax.numpy as jnp
from jax import lax
from jax.experimental import pallas as pl
from jax.experimental.pallas import tpu as pltpu

B, SQ, H, D = 2, 256, 8, 64
BSQ = B * SQ
HD = H * D
BH = B * H
SCALE = D ** -0.5

NC = 8
CH = BSQ // NC
CPB = NC // B
NF = 6
NX = NC + (NC - NF)


def _to_bhsd(x_dense):
    return jnp.transpose(
        x_dense.reshape(B, SQ, H, D), (0, 2, 1, 3)
    ).reshape(BH, SQ, D)


def _to_hsd(x_dense):
    return jnp.transpose(x_dense.reshape(SQ, H, D), (1, 0, 2))


def kernel(Q, K, V):
    Qd = Q.reshape(BSQ, HD)
    Kd = K.reshape(BSQ, HD)
    Vd = V.reshape(BSQ, HD)

    def body(q_ref, k_ref, v_ref, out_ref, loc, rem,
             xs_sems, xr_sems, ys_sems, yr_sems):
        my_x = lax.axis_index("x")
        my_y = lax.axis_index("y")
        x_nbr = (1 - my_x, my_y)
        y_nbr = (my_x, 1 - my_y)
        prim = my_y

        barrier_sem = pltpu.get_barrier_semaphore()
        for nbr in (x_nbr, y_nbr):
            pl.semaphore_signal(
                barrier_sem, inc=1, device_id=nbr,
                device_id_type=pl.DeviceIdType.MESH,
            )

        loc[0, :, :] = k_ref[...].astype(jnp.bfloat16)
        loc[1, :, :] = v_ref[...].astype(jnp.bfloat16)

        pl.semaphore_wait(barrier_sem, 2)
        sec = 1 - prim
        x_plan = [(prim, c) for c in range(NC)] + [
            (sec, c) for c in range(NF, NC)
        ]
        x_rdmas = []
        for i, (t, c) in enumerate(x_plan):
            rdma = pltpu.make_async_remote_copy(
                src_ref=loc.at[t, pl.ds(c * CH, CH)],
                dst_ref=rem.at[t, pl.ds(c * CH, CH)],
                send_sem=xs_sems.at[i],
                recv_sem=xr_sems.at[i],
                device_id=x_nbr,
                device_id_type=pl.DeviceIdType.MESH,
            )
            rdma.start()
            x_rdmas.append(rdma)

        qt = _to_bhsd((q_ref[...] * SCALE).astype(jnp.bfloat16))
        kt1 = _to_bhsd(loc[0, :, :])
        vt1 = _to_bhsd(loc[1, :, :])

        s1 = p1 = l1 = u1 = None
        y_rdmas = []
        for c in range(NF):
            x_rdmas[c].wait_recv()
            rdma = pltpu.make_async_remote_copy(
                src_ref=rem.at[prim, pl.ds(c * CH, CH)],
                dst_ref=rem.at[prim, pl.ds(c * CH, CH)],
                send_sem=ys_sems.at[c],
                recv_sem=yr_sems.at[c],
                device_id=y_nbr,
                device_id_type=pl.DeviceIdType.MESH,
            )
            rdma.start()
            y_rdmas.append(rdma)
            if c == 1:
                s1 = lax.dot_general(
                    qt, kt1, (((2,), (2,)), ((0,), (0,))),
                    preferred_element_type=jnp.float32,
                )
            elif c == 3:
                p1 = jnp.exp(s1)
                l1 = jnp.sum(p1, axis=2, keepdims=True)
            elif c == 5:
                u1 = lax.dot_general(
                    p1.astype(jnp.bfloat16), vt1, (((2,), (1,)), ((0,), (0,))),
                    preferred_element_type=jnp.float32,
                )

        for b in range(B):
            for c in range(b * CPB, (b + 1) * CPB):
                if c < NF:
                    y_rdmas[c].wait_recv()
                else:
                    x_rdmas[NC + c - NF].wait_recv()
            if b == B - 1:
                for c in range(NF, NC):
                    x_rdmas[c].wait_recv()
            kt2 = _to_hsd(rem[0, b * SQ:(b + 1) * SQ, :])
            vt2 = _to_hsd(rem[1, b * SQ:(b + 1) * SQ, :])
            qb = qt[b * H:(b + 1) * H]
            s2 = lax.dot_general(
                qb, kt2, (((2,), (2,)), ((0,), (0,))),
                preferred_element_type=jnp.float32,
            )
            p2 = jnp.exp(s2)
            l2 = jnp.sum(p2, axis=2, keepdims=True)
            u2 = lax.dot_general(
                p2.astype(jnp.bfloat16), vt2, (((2,), (1,)), ((0,), (0,))),
                preferred_element_type=jnp.float32,
            )
            o_b = (u1[b * H:(b + 1) * H] + u2) / (l1[b * H:(b + 1) * H] + l2)
            out_ref[b * SQ:(b + 1) * SQ, :] = jnp.transpose(
                o_b, (1, 0, 2)
            ).reshape(SQ, HD).astype(jnp.bfloat16)

        for r in x_rdmas:
            r.wait_send()
        for r in y_rdmas:
            r.wait_send()

    out_dense = pl.pallas_call(
        body,
        out_shape=jax.ShapeDtypeStruct((BSQ, HD), jnp.bfloat16),
        in_specs=[pl.BlockSpec(memory_space=pltpu.VMEM)] * 3,
        out_specs=pl.BlockSpec(memory_space=pltpu.VMEM),
        scratch_shapes=[
            pltpu.VMEM((2, BSQ, HD), jnp.bfloat16),
            pltpu.VMEM((2, BSQ, HD), jnp.bfloat16),
            pltpu.SemaphoreType.DMA((NX,)),
            pltpu.SemaphoreType.DMA((NX,)),
            pltpu.SemaphoreType.DMA((NF,)),
            pltpu.SemaphoreType.DMA((NF,)),
        ],
        compiler_params=pltpu.CompilerParams(collective_id=0),
    )(Qd, Kd, Vd)
    return out_dense.reshape(B, SQ, H, D)


# device time: 20102 ns/iter; 1.1396x vs baseline; 1.1396x over previous
import jax
import jax.numpy as jnp
from jax import lax
from jax.experimental import pallas as pl
from jax.experimental.pallas import tpu as pltpu

B, SQ, H, D = 2, 256, 8, 64
BSQ = B * SQ
HD = H * D
BH = B * H
SCALE = D ** -0.5

NC = 8
CH = BSQ // NC
CPB = NC // B
NF = 6
NX = NC + (NC - NF)


def _to_bhsd(x_dense):
    return jnp.transpose(
        x_dense.reshape(B, SQ, H, D), (0, 2, 1, 3)
    ).reshape(BH, SQ, D)


def _to_hsd(x_dense):
    return jnp.transpose(x_dense.reshape(SQ, H, D), (1, 0, 2))


def kernel(Q, K, V):
    def body(q_ref, k_ref, v_ref, out_ref, loc, rem,
             xs_sems, xr_sems, ys_sems, yr_sems):
        my_x = lax.axis_index("x")
        my_y = lax.axis_index("y")
        x_nbr = (1 - my_x, my_y)
        y_nbr = (my_x, 1 - my_y)
        prim = my_y

        barrier_sem = pltpu.get_barrier_semaphore()
        for nbr in (x_nbr, y_nbr):
            pl.semaphore_signal(
                barrier_sem, inc=1, device_id=nbr,
                device_id_type=pl.DeviceIdType.MESH,
            )

        loc[0, :, :] = k_ref[...].reshape(BSQ, HD).astype(jnp.bfloat16)
        loc[1, :, :] = v_ref[...].reshape(BSQ, HD).astype(jnp.bfloat16)

        pl.semaphore_wait(barrier_sem, 2)
        sec = 1 - prim
        x_plan = [(prim, c) for c in range(NC)] + [
            (sec, c) for c in range(NF, NC)
        ]
        x_rdmas = []
        for i, (t, c) in enumerate(x_plan):
            rdma = pltpu.make_async_remote_copy(
                src_ref=loc.at[t, pl.ds(c * CH, CH)],
                dst_ref=rem.at[t, pl.ds(c * CH, CH)],
                send_sem=xs_sems.at[i],
                recv_sem=xr_sems.at[i],
                device_id=x_nbr,
                device_id_type=pl.DeviceIdType.MESH,
            )
            rdma.start()
            x_rdmas.append(rdma)

        qt = jnp.transpose(
            (q_ref[...] * SCALE).astype(jnp.bfloat16), (0, 2, 1, 3)
        ).reshape(BH, SQ, D)
        kt1 = _to_bhsd(loc[0, :, :])
        vt1 = _to_bhsd(loc[1, :, :])

        s1 = p1 = l1 = u1 = None
        y_rdmas = []
        for c in range(NF):
            x_rdmas[c].wait_recv()
            rdma = pltpu.make_async_remote_copy(
                src_ref=rem.at[prim, pl.ds(c * CH, CH)],
                dst_ref=rem.at[prim, pl.ds(c * CH, CH)],
                send_sem=ys_sems.at[c],
                recv_sem=yr_sems.at[c],
                device_id=y_nbr,
                device_id_type=pl.DeviceIdType.MESH,
            )
            rdma.start()
            y_rdmas.append(rdma)
            if c == 1:
                s1 = lax.dot_general(
                    qt, kt1, (((2,), (2,)), ((0,), (0,))),
                    preferred_element_type=jnp.float32,
                )
            elif c == 3:
                p1 = jnp.exp(s1)
                l1 = jnp.sum(p1, axis=2, keepdims=True)
            elif c == 5:
                u1 = lax.dot_general(
                    p1.astype(jnp.bfloat16), vt1, (((2,), (1,)), ((0,), (0,))),
                    preferred_element_type=jnp.float32,
                )

        for b in range(B):
            for c in range(b * CPB, (b + 1) * CPB):
                if c < NF:
                    y_rdmas[c].wait_recv()
                else:
                    x_rdmas[NC + c - NF].wait_recv()
            if b == B - 1:
                for c in range(NF, NC):
                    x_rdmas[c].wait_recv()
            kt2 = _to_hsd(rem[0, b * SQ:(b + 1) * SQ, :])
            vt2 = _to_hsd(rem[1, b * SQ:(b + 1) * SQ, :])
            qb = qt[b * H:(b + 1) * H]
            s2 = lax.dot_general(
                qb, kt2, (((2,), (2,)), ((0,), (0,))),
                preferred_element_type=jnp.float32,
            )
            p2 = jnp.exp(s2)
            l2 = jnp.sum(p2, axis=2, keepdims=True)
            u2 = lax.dot_general(
                p2.astype(jnp.bfloat16), vt2, (((2,), (1,)), ((0,), (0,))),
                preferred_element_type=jnp.float32,
            )
            o_b = (u1[b * H:(b + 1) * H] + u2) / (l1[b * H:(b + 1) * H] + l2)
            out_ref[b, :, :, :] = jnp.transpose(
                o_b, (1, 0, 2)
            ).astype(jnp.bfloat16)

        for r in x_rdmas:
            r.wait_send()
        for r in y_rdmas:
            r.wait_send()

    return pl.pallas_call(
        body,
        out_shape=jax.ShapeDtypeStruct((B, SQ, H, D), jnp.bfloat16),
        in_specs=[pl.BlockSpec(memory_space=pltpu.VMEM)] * 3,
        out_specs=pl.BlockSpec(memory_space=pltpu.VMEM),
        scratch_shapes=[
            pltpu.VMEM((2, BSQ, HD), jnp.bfloat16),
            pltpu.VMEM((2, BSQ, HD), jnp.bfloat16),
            pltpu.SemaphoreType.DMA((NX,)),
            pltpu.SemaphoreType.DMA((NX,)),
            pltpu.SemaphoreType.DMA((NF,)),
            pltpu.SemaphoreType.DMA((NF,)),
        ],
        compiler_params=pltpu.CompilerParams(collective_id=0),
    )(Q, K, V)
